# Initial kernel scaffold; baseline (speedup 1.0000x reference)
import jax
import jax.numpy as jnp
from jax.experimental import pallas as pl


def kernel(x, router_w1, router_w2, expert_w1, expert_w2, shared_w1, shared_w2):
    raise NotImplementedError("write your pallas kernel here")


# speedup vs baseline: 1.0075x; 1.0075x over previous
import numpy as np
import jax
import jax.numpy as jnp
from jax.experimental import pallas as pl
from jax.experimental.pallas import tpu as pltpu

BN_EPS_ = 1e-5
_BN_INV = float(1.0 / np.sqrt(1.0 + BN_EPS_))


def _router_consts(C, H, W, ps, P):
    HW = H * W
    S = P * P
    pool = np.zeros((HW, S), np.float32)
    for h in range(H):
        for w in range(W):
            pool[h * W + w, (h // ps) * P + (w // ps)] = 1.0 / (ps * ps)
    shifts = np.zeros((9, S, S), np.float32)
    for d in range(9):
        di, dj = d // 3, d % 3
        for p in range(P):
            for q in range(P):
                sp, sq = p + di - 1, q + dj - 1
                if 0 <= sp < P and 0 <= sq < P:
                    shifts[d, sp * P + sq, p * P + q] = 1.0
    return pool, shifts


def _router_kernel(B, C, E, xall_ref, w1r_ref, w2_ref, pool_ref, shifts_ref,
                   idx_ref, vals_ref):
    xp = jnp.dot(xall_ref[...], pool_ref[...],
                 preferred_element_type=jnp.float32)
    xs = [jnp.dot(xp, shifts_ref[d], preferred_element_type=jnp.float32)
          for d in range(9)]
    w1r = w1r_ref[...]
    w2 = w2_ref[...]
    cols = []
    for b in range(B):
        xcol = jnp.concatenate([xs[d][b * C:(b + 1) * C, :] for d in range(9)],
                               axis=0)
        h = jnp.dot(w1r, xcol, preferred_element_type=jnp.float32) * _BN_INV
        h = h * jax.nn.sigmoid(h)
        lm = jnp.dot(w2, h, preferred_element_type=jnp.float32) * _BN_INV
        cols.append(jnp.mean(lm, axis=1, keepdims=True))
    logits = jnp.concatenate(cols, axis=1)
    m = jnp.max(logits, axis=0, keepdims=True)
    ex = jnp.exp(logits - m)
    probs = ex / jnp.sum(ex, axis=0, keepdims=True)
    ii = jax.lax.broadcasted_iota(jnp.int32, (E, B), 0)
    v1 = jnp.max(probs, axis=0, keepdims=True)
    i1 = jnp.min(jnp.where(probs == v1, ii, E + 1), axis=0, keepdims=True)
    masked = jnp.where(ii == i1, -jnp.inf, probs)
    v2 = jnp.max(masked, axis=0, keepdims=True)
    i2 = jnp.min(jnp.where(masked == v2, ii, E + 1), axis=0, keepdims=True)
    denom = v1 + v2 + 1e-6
    idx_ref[...] = jnp.concatenate([i1, i2], axis=0)
    vals_ref[...] = jnp.concatenate([v1 / denom, v2 / denom], axis=0)


def _expert_kernel(idx_ref, vals_ref, x_ref, w1_ref, w2_ref, sw1_ref, sw2_ref,
                   out_ref):
    b = pl.program_id(0)
    s = pl.program_id(1)
    x2d = x_ref[0]
    h1 = jnp.dot(w1_ref[0], x2d, preferred_element_type=jnp.float32) * _BN_INV
    h1 = h1 * jax.nn.sigmoid(h1)
    eo = jnp.dot(w2_ref[0], h1, preferred_element_type=jnp.float32) * _BN_INV
    contrib = eo * vals_ref[s, b]

    @pl.when(s == 0)
    def _():
        hs = jnp.dot(sw1_ref[...], x2d,
                     preferred_element_type=jnp.float32) * _BN_INV
        hs = hs * jax.nn.sigmoid(hs)
        so = jnp.dot(sw2_ref[...], hs,
                     preferred_element_type=jnp.float32) * _BN_INV
        out_ref[0] = so + contrib

    @pl.when(s != 0)
    def _():
        out_ref[0] = out_ref[0] + contrib


def kernel(x, router_w1, router_w2, expert_w1, expert_w2, shared_w1, shared_w2):
    B, C, H, W = x.shape
    E, red = router_w2.shape
    hid = expert_w1.shape[1]
    HW = H * W
    ps = 4
    P = H // ps
    S = P * P

    pool_np, shifts_np = _router_consts(C, H, W, ps, P)
    pool = jnp.asarray(pool_np)
    shifts = jnp.asarray(shifts_np)
    w1r = jnp.transpose(router_w1, (0, 2, 3, 1)).reshape(red, 9 * C)
    xall = x.reshape(B * C, HW)

    idx, vals = pl.pallas_call(
        lambda *refs: _router_kernel(B, C, E, *refs),
        out_shape=(
            jax.ShapeDtypeStruct((2, B), jnp.int32),
            jax.ShapeDtypeStruct((2, B), jnp.float32),
        ),
    )(xall, w1r, router_w2, pool, shifts)

    x3 = x.reshape(B, C, HW)
    out = pl.pallas_call(
        _expert_kernel,
        grid_spec=pltpu.PrefetchScalarGridSpec(
            num_scalar_prefetch=1,
            grid=(B, 2),
            in_specs=[
                pl.BlockSpec(memory_space=pltpu.SMEM),
                pl.BlockSpec((1, C, HW), lambda b, s, idx: (b, 0, 0)),
                pl.BlockSpec((1, hid, C), lambda b, s, idx: (idx[s, b], 0, 0)),
                pl.BlockSpec((1, C, hid), lambda b, s, idx: (idx[s, b], 0, 0)),
                pl.BlockSpec((hid, C), lambda b, s, idx: (0, 0)),
                pl.BlockSpec((C, hid), lambda b, s, idx: (0, 0)),
            ],
            out_specs=pl.BlockSpec((1, C, HW), lambda b, s, idx: (b, 0, 0)),
        ),
        out_shape=jax.ShapeDtypeStruct((B, C, HW), jnp.float32),
        compiler_params=pltpu.CompilerParams(
            dimension_semantics=("arbitrary", "arbitrary"),
        ),
    )(idx, vals, x3, expert_w1, expert_w2, shared_w1, shared_w2)

    return out.reshape(B, C, H, W)


# speedup vs baseline: 1.0347x; 1.0270x over previous
import numpy as np
import jax
import jax.numpy as jnp
from jax.experimental import pallas as pl
from jax.experimental.pallas import tpu as pltpu

BN_EPS_ = 1e-5
_BN_INV = float(1.0 / np.sqrt(1.0 + BN_EPS_))


def _router_consts(C, H, W, ps, P):
    HW = H * W
    S = P * P
    pool = np.zeros((HW, S), np.float32)
    for h in range(H):
        for w in range(W):
            pool[h * W + w, (h // ps) * P + (w // ps)] = 1.0 / (ps * ps)
    shifts = np.zeros((9, S, S), np.float32)
    for d in range(9):
        di, dj = d // 3, d % 3
        for p in range(P):
            for q in range(P):
                sp, sq = p + di - 1, q + dj - 1
                if 0 <= sp < P and 0 <= sq < P:
                    shifts[d, sp * P + sq, p * P + q] = 1.0
    return pool, shifts


def _router_kernel(B, C, E, xall_ref, w1r_ref, w2_ref, pool_ref, shifts_ref,
                   idx_ref, vals_ref):
    xp = jnp.dot(xall_ref[...], pool_ref[...],
                 preferred_element_type=jnp.float32)
    xs = [jnp.dot(xp, shifts_ref[d], preferred_element_type=jnp.float32)
          for d in range(9)]
    w1r = w1r_ref[...]
    w2 = w2_ref[...]
    cols = []
    for b in range(B):
        xcol = jnp.concatenate([xs[d][b * C:(b + 1) * C, :] for d in range(9)],
                               axis=0)
        h = jnp.dot(w1r, xcol, preferred_element_type=jnp.float32) * _BN_INV
        h = h * jax.nn.sigmoid(h)
        lm = jnp.dot(w2, h, preferred_element_type=jnp.float32) * _BN_INV
        cols.append(jnp.mean(lm, axis=1, keepdims=True))
    logits = jnp.concatenate(cols, axis=1)
    m = jnp.max(logits, axis=0, keepdims=True)
    ex = jnp.exp(logits - m)
    probs = ex / jnp.sum(ex, axis=0, keepdims=True)
    ii = jax.lax.broadcasted_iota(jnp.int32, (E, B), 0)
    v1 = jnp.max(probs, axis=0, keepdims=True)
    i1 = jnp.min(jnp.where(probs == v1, ii, E + 1), axis=0, keepdims=True)
    masked = jnp.where(ii == i1, -jnp.inf, probs)
    v2 = jnp.max(masked, axis=0, keepdims=True)
    i2 = jnp.min(jnp.where(masked == v2, ii, E + 1), axis=0, keepdims=True)
    denom = v1 + v2 + 1e-6
    idx_ref[...] = jnp.concatenate([i1, i2], axis=0)
    vals_ref[...] = jnp.concatenate([v1 / denom, v2 / denom], axis=0)


def _expert_kernel(idx_ref, vals_ref, x_ref, w1_ref, w2_ref, sw1_ref, sw2_ref,
                   out_ref):
    b = pl.program_id(0)
    s = pl.program_id(1)
    x2d = x_ref[0]
    h1 = jnp.dot(w1_ref[0], x2d, preferred_element_type=jnp.float32) * _BN_INV
    h1 = (h1 * jax.nn.sigmoid(h1)).astype(jnp.bfloat16)
    eo = jnp.dot(w2_ref[0], h1, preferred_element_type=jnp.float32) * _BN_INV
    contrib = eo * vals_ref[s, b]

    @pl.when(s == 0)
    def _():
        hs = jnp.dot(sw1_ref[...], x2d,
                     preferred_element_type=jnp.float32) * _BN_INV
        hs = (hs * jax.nn.sigmoid(hs)).astype(jnp.bfloat16)
        so = jnp.dot(sw2_ref[...], hs,
                     preferred_element_type=jnp.float32) * _BN_INV
        out_ref[0] = so + contrib

    @pl.when(s != 0)
    def _():
        out_ref[0] = out_ref[0] + contrib


def kernel(x, router_w1, router_w2, expert_w1, expert_w2, shared_w1, shared_w2):
    B, C, H, W = x.shape
    E, red = router_w2.shape
    hid = expert_w1.shape[1]
    HW = H * W
    ps = 4
    P = H // ps
    S = P * P

    pool_np, shifts_np = _router_consts(C, H, W, ps, P)
    pool = jnp.asarray(pool_np)
    shifts = jnp.asarray(shifts_np)
    w1r = jnp.transpose(router_w1, (0, 2, 3, 1)).reshape(red, 9 * C)
    xall = x.reshape(B * C, HW)

    idx, vals = pl.pallas_call(
        lambda *refs: _router_kernel(B, C, E, *refs),
        out_shape=(
            jax.ShapeDtypeStruct((2, B), jnp.int32),
            jax.ShapeDtypeStruct((2, B), jnp.float32),
        ),
    )(xall, w1r, router_w2, pool, shifts)

    x3 = x.reshape(B, C, HW).astype(jnp.bfloat16)
    ew1 = expert_w1.astype(jnp.bfloat16)
    ew2 = expert_w2.astype(jnp.bfloat16)
    sw1 = shared_w1.astype(jnp.bfloat16)
    sw2 = shared_w2.astype(jnp.bfloat16)
    out = pl.pallas_call(
        _expert_kernel,
        grid_spec=pltpu.PrefetchScalarGridSpec(
            num_scalar_prefetch=1,
            grid=(B, 2),
            in_specs=[
                pl.BlockSpec(memory_space=pltpu.SMEM),
                pl.BlockSpec((1, C, HW), lambda b, s, idx: (b, 0, 0)),
                pl.BlockSpec((1, hid, C), lambda b, s, idx: (idx[s, b], 0, 0)),
                pl.BlockSpec((1, C, hid), lambda b, s, idx: (idx[s, b], 0, 0)),
                pl.BlockSpec((hid, C), lambda b, s, idx: (0, 0)),
                pl.BlockSpec((C, hid), lambda b, s, idx: (0, 0)),
            ],
            out_specs=pl.BlockSpec((1, C, HW), lambda b, s, idx: (b, 0, 0)),
        ),
        out_shape=jax.ShapeDtypeStruct((B, C, HW), jnp.float32),
        compiler_params=pltpu.CompilerParams(
            dimension_semantics=("arbitrary", "arbitrary"),
        ),
    )(idx, vals, x3, ew1, ew2, sw1, sw2)

    return out.reshape(B, C, H, W)


# speedup vs baseline: 1.0636x; 1.0279x over previous
import numpy as np
import jax
import jax.numpy as jnp
from jax.experimental import pallas as pl
from jax.experimental.pallas import tpu as pltpu

BN_EPS_ = 1e-5
_BN_INV = float(1.0 / np.sqrt(1.0 + BN_EPS_))


def _router_consts(C, H, W, ps, P):
    HW = H * W
    S = P * P
    pool = np.zeros((HW, S), np.float32)
    for h in range(H):
        for w in range(W):
            pool[h * W + w, (h // ps) * P + (w // ps)] = 1.0 / (ps * ps)
    shifts = np.zeros((9, S, S), np.float32)
    for d in range(9):
        di, dj = d // 3, d % 3
        for p in range(P):
            for q in range(P):
                sp, sq = p + di - 1, q + dj - 1
                if 0 <= sp < P and 0 <= sq < P:
                    shifts[d, sp * P + sq, p * P + q] = 1.0
    return pool, shifts


def _router_kernel(B, C, E, xall_ref, w1r_ref, w2_ref, pool_ref, shifts_ref,
                   idx_ref, vals_ref):
    xp = jnp.dot(xall_ref[...], pool_ref[...],
                 preferred_element_type=jnp.float32)
    xs = [jnp.dot(xp, shifts_ref[d], preferred_element_type=jnp.float32)
          for d in range(9)]
    w1r = w1r_ref[...]
    w2 = w2_ref[...]
    cols = []
    for b in range(B):
        xcol = jnp.concatenate([xs[d][b * C:(b + 1) * C, :] for d in range(9)],
                               axis=0)
        h = jnp.dot(w1r, xcol, preferred_element_type=jnp.float32)
        h = h * jax.nn.sigmoid(h)
        lm = jnp.dot(w2, h, preferred_element_type=jnp.float32)
        cols.append(jnp.mean(lm, axis=1, keepdims=True))
    logits = jnp.concatenate(cols, axis=1)
    m = jnp.max(logits, axis=0, keepdims=True)
    ex = jnp.exp(logits - m)
    probs = ex / jnp.sum(ex, axis=0, keepdims=True)
    ii = jax.lax.broadcasted_iota(jnp.int32, (E, B), 0)
    v1 = jnp.max(probs, axis=0, keepdims=True)
    i1 = jnp.min(jnp.where(probs == v1, ii, E + 1), axis=0, keepdims=True)
    masked = jnp.where(ii == i1, -jnp.inf, probs)
    v2 = jnp.max(masked, axis=0, keepdims=True)
    i2 = jnp.min(jnp.where(masked == v2, ii, E + 1), axis=0, keepdims=True)
    denom = v1 + v2 + 1e-6
    idx_ref[...] = jnp.concatenate([i1, i2], axis=0)
    vals_ref[...] = jnp.concatenate([v1 / denom, v2 / denom], axis=0)


def _expert_kernel(idx_ref, vals_ref, x_ref, ew1_ref, ew2_ref, sw1_ref,
                   sw2_ref, out_ref):
    b = pl.program_id(0)
    e0 = idx_ref[0, b]
    e1 = idx_ref[1, b]
    va = vals_ref[0, b]
    vb = vals_ref[1, b]
    xb = x_ref[0].astype(jnp.bfloat16)

    def expert(w1, w2):
        h1 = jnp.dot(w1, xb, preferred_element_type=jnp.float32)
        h1 = (h1 * jax.nn.sigmoid(h1)).astype(jnp.bfloat16)
        return jnp.dot(w2, h1, preferred_element_type=jnp.float32)

    eo_a = expert(ew1_ref[e0], ew2_ref[e0])
    eo_b = expert(ew1_ref[e1], ew2_ref[e1])
    so = expert(sw1_ref[...], sw2_ref[...])
    out_ref[0] = va * eo_a + vb * eo_b + so


def kernel(x, router_w1, router_w2, expert_w1, expert_w2, shared_w1, shared_w2):
    B, C, H, W = x.shape
    E, red = router_w2.shape
    hid = expert_w1.shape[1]
    HW = H * W
    ps = 4
    P = H // ps

    pool_np, shifts_np = _router_consts(C, H, W, ps, P)
    pool = jnp.asarray(pool_np)
    shifts = jnp.asarray(shifts_np)
    w1r = jnp.transpose(router_w1, (0, 2, 3, 1)).reshape(red, 9 * C) * _BN_INV
    rw2 = router_w2 * _BN_INV
    xall = x.reshape(B * C, HW)

    idx, vals = pl.pallas_call(
        lambda *refs: _router_kernel(B, C, E, *refs),
        out_shape=(
            jax.ShapeDtypeStruct((2, B), jnp.int32),
            jax.ShapeDtypeStruct((2, B), jnp.float32),
        ),
    )(xall, w1r, rw2, pool, shifts)

    x3 = x.reshape(B, C, HW)
    ew1 = (expert_w1 * _BN_INV).astype(jnp.bfloat16)
    ew2 = (expert_w2 * _BN_INV).astype(jnp.bfloat16)
    sw1 = (shared_w1 * _BN_INV).astype(jnp.bfloat16)
    sw2 = (shared_w2 * _BN_INV).astype(jnp.bfloat16)

    out = pl.pallas_call(
        _expert_kernel,
        grid=(B,),
        in_specs=[
            pl.BlockSpec(memory_space=pltpu.SMEM),
            pl.BlockSpec(memory_space=pltpu.SMEM),
            pl.BlockSpec((1, C, HW), lambda b: (b, 0, 0)),
            pl.BlockSpec((E, hid, C), lambda b: (0, 0, 0)),
            pl.BlockSpec((E, C, hid), lambda b: (0, 0, 0)),
            pl.BlockSpec((hid, C), lambda b: (0, 0)),
            pl.BlockSpec((C, hid), lambda b: (0, 0)),
        ],
        out_specs=pl.BlockSpec((1, C, HW), lambda b: (b, 0, 0)),
        out_shape=jax.ShapeDtypeStruct((B, C, HW), jnp.float32),
    )(idx, vals, x3, ew1, ew2, sw1, sw2)

    return out.reshape(B, C, H, W)


# speedup vs baseline: 1.5041x; 1.4142x over previous
import numpy as np
import jax
import jax.numpy as jnp
from jax.experimental import pallas as pl
from jax.experimental.pallas import tpu as pltpu

BN_EPS_ = 1e-5
_BN_INV = float(1.0 / np.sqrt(1.0 + BN_EPS_))


def _router_consts(C, H, W, ps, P):
    HW = H * W
    S = P * P
    pool = np.zeros((HW, S), np.float32)
    for h in range(H):
        for w in range(W):
            pool[h * W + w, (h // ps) * P + (w // ps)] = 1.0 / (ps * ps)
    shifts = np.zeros((9, S, S), np.float32)
    for d in range(9):
        di, dj = d // 3, d % 3
        for p in range(P):
            for q in range(P):
                sp, sq = p + di - 1, q + dj - 1
                if 0 <= sp < P and 0 <= sq < P:
                    shifts[d, sp * P + sq, p * P + q] = 1.0
    return pool, shifts


def _fused_kernel(C, E, x_ref, pool_ref, shifts_ref, w1r_ref, rw2_ref,
                  ew1_ref, ew2_ref, sw1_ref, sw2_ref, out_ref):
    xb = x_ref[0]
    xp = jnp.dot(xb, pool_ref[...], preferred_element_type=jnp.float32)
    xcol = jnp.concatenate(
        [jnp.dot(xp, shifts_ref[d], preferred_element_type=jnp.float32)
         for d in range(9)], axis=0)
    h = jnp.dot(w1r_ref[...], xcol,
                preferred_element_type=jnp.float32) * _BN_INV
    h = h * jax.nn.sigmoid(h)
    lm = jnp.dot(rw2_ref[...], h,
                 preferred_element_type=jnp.float32) * _BN_INV
    logits = jnp.mean(lm, axis=1, keepdims=True)
    m = jnp.max(logits)
    ex = jnp.exp(logits - m)
    probs = ex / jnp.sum(ex)
    fi = jax.lax.broadcasted_iota(jnp.int32, (E, 1), 0).astype(jnp.float32)
    v1 = jnp.max(probs)
    e0f = jnp.min(jnp.where(probs == v1, fi, float(E + 1)))
    masked = jnp.where(fi == e0f, -jnp.inf, probs)
    v2 = jnp.max(masked)
    e1f = jnp.min(jnp.where(masked == v2, fi, float(E + 1)))
    e0 = e0f.astype(jnp.int32)
    e1 = e1f.astype(jnp.int32)
    denom = v1 + v2 + 1e-6
    va = v1 / denom * _BN_INV
    vb = v2 / denom * _BN_INV

    xbf = (xb * _BN_INV).astype(jnp.bfloat16)

    def expert(w1, w2):
        h1 = jnp.dot(w1, xbf, preferred_element_type=jnp.float32)
        h1 = (h1 * jax.nn.sigmoid(h1)).astype(jnp.bfloat16)
        return jnp.dot(w2, h1, preferred_element_type=jnp.float32)

    eo_a = expert(ew1_ref[e0], ew2_ref[e0])
    eo_b = expert(ew1_ref[e1], ew2_ref[e1])
    so = expert(sw1_ref[...], sw2_ref[...])
    out_ref[0] = va * eo_a + vb * eo_b + _BN_INV * so


def kernel(x, router_w1, router_w2, expert_w1, expert_w2, shared_w1, shared_w2):
    B, C, H, W = x.shape
    E, red = router_w2.shape
    hid = expert_w1.shape[1]
    HW = H * W
    ps = 4
    P = H // ps
    S = P * P

    pool_np, shifts_np = _router_consts(C, H, W, ps, P)
    pool = jnp.asarray(pool_np)
    shifts = jnp.asarray(shifts_np)
    w1r = jnp.transpose(router_w1, (0, 2, 3, 1)).reshape(red, 9 * C)
    x3 = x.reshape(B, C, HW)
    ew1 = expert_w1.astype(jnp.bfloat16)
    ew2 = expert_w2.astype(jnp.bfloat16)
    sw1 = shared_w1.astype(jnp.bfloat16)
    sw2 = shared_w2.astype(jnp.bfloat16)

    out = pl.pallas_call(
        lambda *refs: _fused_kernel(C, E, *refs),
        grid=(B,),
        in_specs=[
            pl.BlockSpec((1, C, HW), lambda b: (b, 0, 0)),
            pl.BlockSpec((HW, S), lambda b: (0, 0)),
            pl.BlockSpec((9, S, S), lambda b: (0, 0, 0)),
            pl.BlockSpec((red, 9 * C), lambda b: (0, 0)),
            pl.BlockSpec((E, red), lambda b: (0, 0)),
            pl.BlockSpec((E, hid, C), lambda b: (0, 0, 0)),
            pl.BlockSpec((E, C, hid), lambda b: (0, 0, 0)),
            pl.BlockSpec((hid, C), lambda b: (0, 0)),
            pl.BlockSpec((C, hid), lambda b: (0, 0)),
        ],
        out_specs=pl.BlockSpec((1, C, HW), lambda b: (b, 0, 0)),
        out_shape=jax.ShapeDtypeStruct((B, C, HW), jnp.float32),
    )(x3, pool, shifts, w1r, router_w2, ew1, ew2, sw1, sw2)

    return out.reshape(B, C, H, W)


# speedup vs baseline: 1.7088x; 1.1361x over previous
import numpy as np
import jax
import jax.numpy as jnp
from jax.experimental import pallas as pl
from jax.experimental.pallas import tpu as pltpu

BN_EPS_ = 1e-5
_BN_INV = float(1.0 / np.sqrt(1.0 + BN_EPS_))


def _router_consts(C, H, W, ps, P):
    HW = H * W
    S = P * P
    pool = np.zeros((HW, S), np.float32)
    for h in range(H):
        for w in range(W):
            pool[h * W + w, (h // ps) * P + (w // ps)] = 1.0 / (ps * ps)
    shifts = np.zeros((9, S, S), np.float32)
    for d in range(9):
        di, dj = d // 3, d % 3
        for p in range(P):
            for q in range(P):
                sp, sq = p + di - 1, q + dj - 1
                if 0 <= sp < P and 0 <= sq < P:
                    shifts[d, sp * P + sq, p * P + q] = 1.0
    return pool, shifts


def _fused_kernel(C, E, NB, x_ref, pool_ref, shifts_ref, w1r_ref, rw2_ref,
                  ew1_ref, ew2_ref, sw1_ref, sw2_ref, out_ref):
    probs_l = []
    xbf_l = []
    for j in range(NB):
        xb = x_ref[j]
        xbf_l.append((xb * _BN_INV).astype(jnp.bfloat16))
        xp = jnp.dot(xb, pool_ref[...], preferred_element_type=jnp.float32)
        xcol = jnp.concatenate(
            [jnp.dot(xp, shifts_ref[d], preferred_element_type=jnp.float32)
             for d in range(9)], axis=0)
        h = jnp.dot(w1r_ref[...], xcol,
                    preferred_element_type=jnp.float32) * _BN_INV
        h = h * jax.nn.sigmoid(h)
        lm = jnp.dot(rw2_ref[...], h,
                     preferred_element_type=jnp.float32) * _BN_INV
        logits = jnp.mean(lm, axis=1, keepdims=True)
        m = jnp.max(logits)
        ex = jnp.exp(logits - m)
        probs_l.append(ex / jnp.sum(ex))

    sel = []
    fi = jax.lax.broadcasted_iota(jnp.int32, (E, 1), 0).astype(jnp.float32)
    for j in range(NB):
        probs = probs_l[j]
        v1 = jnp.max(probs)
        e0f = jnp.min(jnp.where(probs == v1, fi, float(E + 1)))
        masked = jnp.where(fi == e0f, -jnp.inf, probs)
        v2 = jnp.max(masked)
        e1f = jnp.min(jnp.where(masked == v2, fi, float(E + 1)))
        denom = v1 + v2 + 1e-6
        sel.append((e0f.astype(jnp.int32), e1f.astype(jnp.int32),
                    v1 / denom * _BN_INV, v2 / denom * _BN_INV))

    h1_l = []
    for j in range(NB):
        e0, e1, _, _ = sel[j]
        for w1 in (ew1_ref[e0], ew1_ref[e1], sw1_ref[...]):
            h1 = jnp.dot(w1, xbf_l[j], preferred_element_type=jnp.float32)
            h1_l.append((h1 * jax.nn.sigmoid(h1)).astype(jnp.bfloat16))

    for j in range(NB):
        e0, e1, va, vb = sel[j]
        eo_a = jnp.dot(ew2_ref[e0], h1_l[3 * j + 0],
                       preferred_element_type=jnp.float32)
        eo_b = jnp.dot(ew2_ref[e1], h1_l[3 * j + 1],
                       preferred_element_type=jnp.float32)
        so = jnp.dot(sw2_ref[...], h1_l[3 * j + 2],
                     preferred_element_type=jnp.float32)
        out_ref[j] = va * eo_a + vb * eo_b + _BN_INV * so


def kernel(x, router_w1, router_w2, expert_w1, expert_w2, shared_w1, shared_w2):
    B, C, H, W = x.shape
    E, red = router_w2.shape
    hid = expert_w1.shape[1]
    HW = H * W
    ps = 4
    P = H // ps
    S = P * P

    pool_np, shifts_np = _router_consts(C, H, W, ps, P)
    pool = jnp.asarray(pool_np)
    shifts = jnp.asarray(shifts_np)
    w1r = jnp.transpose(router_w1, (0, 2, 3, 1)).reshape(red, 9 * C)
    x3 = x.reshape(B, C, HW)
    ew1 = expert_w1.astype(jnp.bfloat16)
    ew2 = expert_w2.astype(jnp.bfloat16)
    sw1 = shared_w1.astype(jnp.bfloat16)
    sw2 = shared_w2.astype(jnp.bfloat16)

    NB = 2
    out = pl.pallas_call(
        lambda *refs: _fused_kernel(C, E, NB, *refs),
        grid=(B // NB,),
        in_specs=[
            pl.BlockSpec((NB, C, HW), lambda b: (b, 0, 0)),
            pl.BlockSpec((HW, S), lambda b: (0, 0)),
            pl.BlockSpec((9, S, S), lambda b: (0, 0, 0)),
            pl.BlockSpec((red, 9 * C), lambda b: (0, 0)),
            pl.BlockSpec((E, red), lambda b: (0, 0)),
            pl.BlockSpec((E, hid, C), lambda b: (0, 0, 0)),
            pl.BlockSpec((E, C, hid), lambda b: (0, 0, 0)),
            pl.BlockSpec((hid, C), lambda b: (0, 0)),
            pl.BlockSpec((C, hid), lambda b: (0, 0)),
        ],
        out_specs=pl.BlockSpec((NB, C, HW), lambda b: (b, 0, 0)),
        out_shape=jax.ShapeDtypeStruct((B, C, HW), jnp.float32),
    )(x3, pool, shifts, w1r, router_w2, ew1, ew2, sw1, sw2)

    return out.reshape(B, C, H, W)


# speedup vs baseline: 1.7464x; 1.0220x over previous
import numpy as np
import jax
import jax.numpy as jnp
from jax.experimental import pallas as pl
from jax.experimental.pallas import tpu as pltpu

BN_EPS_ = 1e-5
_BN_INV = float(1.0 / np.sqrt(1.0 + BN_EPS_))


def _router_consts(C, H, W, ps, P):
    HW = H * W
    S = P * P
    pool = np.zeros((HW, S), np.float32)
    for h in range(H):
        for w in range(W):
            pool[h * W + w, (h // ps) * P + (w // ps)] = 1.0 / (ps * ps)
    shifts = np.zeros((9, S, S), np.float32)
    for d in range(9):
        di, dj = d // 3, d % 3
        for p in range(P):
            for q in range(P):
                sp, sq = p + di - 1, q + dj - 1
                if 0 <= sp < P and 0 <= sq < P:
                    shifts[d, sp * P + sq, p * P + q] = 1.0
    return pool, shifts


def _fused_kernel(C, E, NB, x_ref, pool_ref, shifts_ref, w1r_ref, rw2_ref,
                  ew1_ref, ew2_ref, sw1_ref, sw2_ref, out_ref):
    probs_l = []
    xbf_l = []
    for j in range(NB):
        xb = x_ref[j]
        xbf_l.append((xb * _BN_INV).astype(jnp.bfloat16))
        xp = jnp.dot(xb, pool_ref[...], preferred_element_type=jnp.float32)
        xcol = jnp.concatenate(
            [jnp.dot(xp, shifts_ref[d], preferred_element_type=jnp.float32)
             for d in range(9)], axis=0)
        h = jnp.dot(w1r_ref[...], xcol,
                    preferred_element_type=jnp.float32) * _BN_INV
        h = h * jax.nn.sigmoid(h)
        lm = jnp.dot(rw2_ref[...], h,
                     preferred_element_type=jnp.float32) * _BN_INV
        logits = jnp.mean(lm, axis=1, keepdims=True)
        m = jnp.max(logits)
        ex = jnp.exp(logits - m)
        probs_l.append(ex / jnp.sum(ex))

    sel = []
    fi = jax.lax.broadcasted_iota(jnp.int32, (E, 1), 0).astype(jnp.float32)
    for j in range(NB):
        probs = probs_l[j]
        v1 = jnp.max(probs)
        e0f = jnp.min(jnp.where(probs == v1, fi, float(E + 1)))
        masked = jnp.where(fi == e0f, -jnp.inf, probs)
        v2 = jnp.max(masked)
        e1f = jnp.min(jnp.where(masked == v2, fi, float(E + 1)))
        denom = v1 + v2 + 1e-6
        sel.append((e0f.astype(jnp.int32), e1f.astype(jnp.int32),
                    v1 / denom * _BN_INV, v2 / denom * _BN_INV))

    h1_l = []
    for j in range(NB):
        e0, e1, _, _ = sel[j]
        for w1 in (ew1_ref[e0], ew1_ref[e1], sw1_ref[...]):
            h1 = jnp.dot(w1, xbf_l[j], preferred_element_type=jnp.float32)
            h1_l.append((h1 * jax.nn.sigmoid(h1)).astype(jnp.bfloat16))

    for j in range(NB):
        e0, e1, va, vb = sel[j]
        eo_a = jnp.dot(ew2_ref[e0], h1_l[3 * j + 0],
                       preferred_element_type=jnp.float32)
        eo_b = jnp.dot(ew2_ref[e1], h1_l[3 * j + 1],
                       preferred_element_type=jnp.float32)
        so = jnp.dot(sw2_ref[...], h1_l[3 * j + 2],
                     preferred_element_type=jnp.float32)
        out_ref[j] = va * eo_a + vb * eo_b + _BN_INV * so


def kernel(x, router_w1, router_w2, expert_w1, expert_w2, shared_w1, shared_w2):
    B, C, H, W = x.shape
    E, red = router_w2.shape
    hid = expert_w1.shape[1]
    HW = H * W
    ps = 4
    P = H // ps
    S = P * P

    pool_np, shifts_np = _router_consts(C, H, W, ps, P)
    pool = jnp.asarray(pool_np)
    shifts = jnp.asarray(shifts_np)
    w1r = jnp.transpose(router_w1, (0, 2, 3, 1)).reshape(red, 9 * C)
    x3 = x.reshape(B, C, HW)
    ew1 = expert_w1.astype(jnp.bfloat16)
    ew2 = expert_w2.astype(jnp.bfloat16)
    sw1 = shared_w1.astype(jnp.bfloat16)
    sw2 = shared_w2.astype(jnp.bfloat16)

    NB = 4
    out = pl.pallas_call(
        lambda *refs: _fused_kernel(C, E, NB, *refs),
        grid=(B // NB,),
        in_specs=[
            pl.BlockSpec((NB, C, HW), lambda b: (b, 0, 0)),
            pl.BlockSpec((HW, S), lambda b: (0, 0)),
            pl.BlockSpec((9, S, S), lambda b: (0, 0, 0)),
            pl.BlockSpec((red, 9 * C), lambda b: (0, 0)),
            pl.BlockSpec((E, red), lambda b: (0, 0)),
            pl.BlockSpec((E, hid, C), lambda b: (0, 0, 0)),
            pl.BlockSpec((E, C, hid), lambda b: (0, 0, 0)),
            pl.BlockSpec((hid, C), lambda b: (0, 0)),
            pl.BlockSpec((C, hid), lambda b: (0, 0)),
        ],
        out_specs=pl.BlockSpec((NB, C, HW), lambda b: (b, 0, 0)),
        out_shape=jax.ShapeDtypeStruct((B, C, HW), jnp.float32),
    )(x3, pool, shifts, w1r, router_w2, ew1, ew2, sw1, sw2)

    return out.reshape(B, C, H, W)


# speedup vs baseline: 1.7924x; 1.0264x over previous
import numpy as np
import jax
import jax.numpy as jnp
from jax.experimental import pallas as pl
from jax.experimental.pallas import tpu as pltpu

BN_EPS_ = 1e-5
_BN_INV = float(1.0 / np.sqrt(1.0 + BN_EPS_))


def _router_consts(C, H, W, ps, P):
    HW = H * W
    S = P * P
    pool = np.zeros((HW, S), np.float32)
    for h in range(H):
        for w in range(W):
            pool[h * W + w, (h // ps) * P + (w // ps)] = 1.0 / (ps * ps)
    shifts = np.zeros((9, S, S), np.float32)
    for d in range(9):
        di, dj = d // 3, d % 3
        for p in range(P):
            for q in range(P):
                sp, sq = p + di - 1, q + dj - 1
                if 0 <= sp < P and 0 <= sq < P:
                    shifts[d, sp * P + sq, p * P + q] = 1.0
    return pool, shifts


def _fused_kernel(C, E, NB, x_ref, pool_ref, shifts_ref, w1r_ref, rw2_ref,
                  ew1f_ref, ew2f_ref, sw1f_ref, sw2f_ref, out_ref,
                  ew1_ref, ew2_ref, sw1_ref, sw2_ref):
    @pl.when(pl.program_id(0) == 0)
    def _():
        for e in range(E):
            ew1_ref[e] = ew1f_ref[e].astype(jnp.bfloat16)
            ew2_ref[e] = ew2f_ref[e].astype(jnp.bfloat16)
        sw1_ref[...] = sw1f_ref[...].astype(jnp.bfloat16)
        sw2_ref[...] = sw2f_ref[...].astype(jnp.bfloat16)

    probs_l = []
    xbf_l = []
    for j in range(NB):
        xb = x_ref[j]
        xbf_l.append((xb * _BN_INV).astype(jnp.bfloat16))
        xp = jnp.dot(xb, pool_ref[...], preferred_element_type=jnp.float32)
        xcol = jnp.concatenate(
            [jnp.dot(xp, shifts_ref[d], preferred_element_type=jnp.float32)
             for d in range(9)], axis=0)
        h = jnp.dot(w1r_ref[...], xcol,
                    preferred_element_type=jnp.float32) * _BN_INV
        h = h * jax.nn.sigmoid(h)
        lm = jnp.dot(rw2_ref[...], h,
                     preferred_element_type=jnp.float32) * _BN_INV
        logits = jnp.mean(lm, axis=1, keepdims=True)
        m = jnp.max(logits)
        ex = jnp.exp(logits - m)
        probs_l.append(ex / jnp.sum(ex))

    sel = []
    fi = jax.lax.broadcasted_iota(jnp.int32, (E, 1), 0).astype(jnp.float32)
    for j in range(NB):
        probs = probs_l[j]
        v1 = jnp.max(probs)
        e0f = jnp.min(jnp.where(probs == v1, fi, float(E + 1)))
        masked = jnp.where(fi == e0f, -jnp.inf, probs)
        v2 = jnp.max(masked)
        e1f = jnp.min(jnp.where(masked == v2, fi, float(E + 1)))
        denom = v1 + v2 + 1e-6
        sel.append((e0f.astype(jnp.int32), e1f.astype(jnp.int32),
                    v1 / denom * _BN_INV, v2 / denom * _BN_INV))

    h1_l = []
    for j in range(NB):
        e0, e1, _, _ = sel[j]
        for w1 in (ew1_ref[e0], ew1_ref[e1], sw1_ref[...]):
            h1 = jnp.dot(w1, xbf_l[j], preferred_element_type=jnp.float32)
            h1_l.append((h1 * jax.nn.sigmoid(h1)).astype(jnp.bfloat16))

    for j in range(NB):
        e0, e1, va, vb = sel[j]
        eo_a = jnp.dot(ew2_ref[e0], h1_l[3 * j + 0],
                       preferred_element_type=jnp.float32)
        eo_b = jnp.dot(ew2_ref[e1], h1_l[3 * j + 1],
                       preferred_element_type=jnp.float32)
        so = jnp.dot(sw2_ref[...], h1_l[3 * j + 2],
                     preferred_element_type=jnp.float32)
        out_ref[j] = va * eo_a + vb * eo_b + _BN_INV * so


def kernel(x, router_w1, router_w2, expert_w1, expert_w2, shared_w1, shared_w2):
    B, C, H, W = x.shape
    E, red = router_w2.shape
    hid = expert_w1.shape[1]
    HW = H * W
    ps = 4
    P = H // ps
    S = P * P

    pool_np, shifts_np = _router_consts(C, H, W, ps, P)
    pool = jnp.asarray(pool_np)
    shifts = jnp.asarray(shifts_np)
    w1r = jnp.transpose(router_w1, (0, 2, 3, 1)).reshape(red, 9 * C)
    x3 = x.reshape(B, C, HW)

    NB = 4
    out = pl.pallas_call(
        lambda *refs: _fused_kernel(C, E, NB, *refs),
        grid=(B // NB,),
        in_specs=[
            pl.BlockSpec((NB, C, HW), lambda b: (b, 0, 0)),
            pl.BlockSpec((HW, S), lambda b: (0, 0)),
            pl.BlockSpec((9, S, S), lambda b: (0, 0, 0)),
            pl.BlockSpec((red, 9 * C), lambda b: (0, 0)),
            pl.BlockSpec((E, red), lambda b: (0, 0)),
            pl.BlockSpec((E, hid, C), lambda b: (0, 0, 0)),
            pl.BlockSpec((E, C, hid), lambda b: (0, 0, 0)),
            pl.BlockSpec((hid, C), lambda b: (0, 0)),
            pl.BlockSpec((C, hid), lambda b: (0, 0)),
        ],
        out_specs=pl.BlockSpec((NB, C, HW), lambda b: (b, 0, 0)),
        out_shape=jax.ShapeDtypeStruct((B, C, HW), jnp.float32),
        scratch_shapes=[
            pltpu.VMEM((E, hid, C), jnp.bfloat16),
            pltpu.VMEM((E, C, hid), jnp.bfloat16),
            pltpu.VMEM((hid, C), jnp.bfloat16),
            pltpu.VMEM((C, hid), jnp.bfloat16),
        ],
    )(x3, pool, shifts, w1r, router_w2, expert_w1, expert_w2,
      shared_w1, shared_w2)

    return out.reshape(B, C, H, W)
